# Initial kernel scaffold; baseline (speedup 1.0000x reference)
import jax
import jax.numpy as jnp
from jax.experimental import pallas as pl


def kernel(x, W, b, codebooks):
    raise NotImplementedError("write your pallas kernel here")


# speedup vs baseline: 5.0399x; 5.0399x over previous
import functools

import jax
import jax.numpy as jnp
from jax import lax
from jax.experimental import pallas as pl
from jax.experimental.pallas import tpu as pltpu
from jax.experimental.pallas import tpu_sc as plsc

K = 1024
D = 256
Q = 4
N = 32768
BN = 1024
NB = N // BN
NC = 2
NS = 16
NW = NC * NS
CHUNK = N // NW
LANES = 16

_PREC = lax.Precision.DEFAULT


def _argmin_rows(dist):
    m = jnp.min(dist, axis=1, keepdims=True)
    iot = lax.broadcasted_iota(jnp.int32, dist.shape, 1)
    idx = jnp.min(jnp.where(dist == m, iot, jnp.int32(K)), axis=1)
    return idx, m[:, 0]


def _cdist_code_kernel(y_ref, a2_ref, cb_ref, b2_ref, codes_ref, rowd2_ref):
    y = y_ref[...]
    a2 = a2_ref[0, 0, :]
    cb = cb_ref[...]
    b2 = b2_ref[0, 0, :]
    ab = lax.dot_general(y, cb, (((1,), (1,)), ((), ())),
                         precision=_PREC)
    d2 = (a2[:, None] + b2[None, :]) - 2.0 * ab
    dist = jnp.sqrt(jnp.maximum(d2, 0.0))
    idx, _ = _argmin_rows(dist)
    codes_ref[0, 0, :] = idx
    rowd2_ref[0, 0, :] = jnp.maximum(jnp.min(d2, axis=1), 0.0)


def _mapping_kernel(a_ref, b_ref, a2_ref, b2_ref, map_ref, md2_ref):
    a = a_ref[0]
    bm = b_ref[0]
    a2 = a2_ref[0, 0, :]
    b2 = b2_ref[0, 0, :]
    ab = lax.dot_general(a, bm, (((1,), (1,)), ((), ())),
                         precision=_PREC)
    d2 = (a2[:, None] + b2[None, :]) - 2.0 * ab
    dist = jnp.sqrt(jnp.maximum(d2, 0.0))
    idx, _ = _argmin_rows(dist)
    map_ref[0, 0, :] = idx
    md2_ref[0, 0, :] = jnp.maximum(jnp.min(d2, axis=1), 0.0)


def _sc_chain_kernel(codes0_hbm, rowd2_hbm, maps_hbm, md2_hbm,
                     codesq_hbm, occ_hbm, accs_hbm,
                     codes_v, rowd2_v, m1_v, m2_v, m3_v, d1_v, d2_v, d3_v,
                     c1_v, c2_v, c3_v, occ0_v, occ1_v, occ2_v, occ3_v,
                     acc_v):
    wid = lax.axis_index("s") * NC + lax.axis_index("c")
    base = wid * CHUNK

    pltpu.sync_copy(codes0_hbm.at[pl.ds(base, CHUNK)], codes_v)
    pltpu.sync_copy(rowd2_hbm.at[pl.ds(base, CHUNK)], rowd2_v)
    pltpu.sync_copy(maps_hbm.at[pl.ds(0, K)], m1_v)
    pltpu.sync_copy(maps_hbm.at[pl.ds(K, K)], m2_v)
    pltpu.sync_copy(maps_hbm.at[pl.ds(2 * K, K)], m3_v)
    pltpu.sync_copy(md2_hbm.at[pl.ds(0, K)], d1_v)
    pltpu.sync_copy(md2_hbm.at[pl.ds(K, K)], d2_v)
    pltpu.sync_copy(md2_hbm.at[pl.ds(2 * K, K)], d3_v)

    zeros = jnp.zeros((LANES,), jnp.int32)
    for j in range(K // LANES):
        sl = pl.ds(j * LANES, LANES)
        occ0_v[sl] = zeros
        occ1_v[sl] = zeros
        occ2_v[sl] = zeros
        occ3_v[sl] = zeros

    ones = jnp.ones((LANES,), jnp.int32)
    acc0 = jnp.zeros((LANES,), jnp.float32)
    acc1 = jnp.zeros((LANES,), jnp.float32)
    acc2 = jnp.zeros((LANES,), jnp.float32)
    acc3 = jnp.zeros((LANES,), jnp.float32)
    for j in range(CHUNK // LANES):
        sl = pl.ds(j * LANES, LANES)
        v0 = codes_v[sl]
        acc0 = acc0 + rowd2_v[sl]
        c1 = plsc.load_gather(m1_v, [v0])
        c2 = plsc.load_gather(m2_v, [c1])
        c3 = plsc.load_gather(m3_v, [c2])
        acc1 = acc1 + plsc.load_gather(d1_v, [v0])
        acc2 = acc2 + plsc.load_gather(d2_v, [c1])
        acc3 = acc3 + plsc.load_gather(d3_v, [c2])
        c1_v[sl] = c1
        c2_v[sl] = c2
        c3_v[sl] = c3
        plsc.store_scatter(occ0_v, [v0], ones)
        plsc.store_scatter(occ1_v, [c1], ones)
        plsc.store_scatter(occ2_v, [c2], ones)
        plsc.store_scatter(occ3_v, [c3], ones)

    acc_v[pl.ds(0, LANES)] = acc0
    acc_v[pl.ds(LANES, LANES)] = acc1
    acc_v[pl.ds(2 * LANES, LANES)] = acc2
    acc_v[pl.ds(3 * LANES, LANES)] = acc3

    pltpu.sync_copy(c1_v, codesq_hbm.at[pl.ds(0 * N + base, CHUNK)])
    pltpu.sync_copy(c2_v, codesq_hbm.at[pl.ds(1 * N + base, CHUNK)])
    pltpu.sync_copy(c3_v, codesq_hbm.at[pl.ds(2 * N + base, CHUNK)])
    obase = wid * Q * K
    pltpu.sync_copy(occ0_v, occ_hbm.at[pl.ds(obase, K)])
    pltpu.sync_copy(occ1_v, occ_hbm.at[pl.ds(obase + K, K)])
    pltpu.sync_copy(occ2_v, occ_hbm.at[pl.ds(obase + 2 * K, K)])
    pltpu.sync_copy(occ3_v, occ_hbm.at[pl.ds(obase + 3 * K, K)])
    pltpu.sync_copy(acc_v, accs_hbm.at[pl.ds(wid * Q * LANES, Q * LANES)])


def _scalars_kernel(accs_ref, occ_ref, loss_ref, perp_ref, used_ref):
    accs = accs_ref[...]
    sums = jnp.sum(jnp.sum(accs, axis=0), axis=1)
    mse = sums / jnp.float32(N * D)
    loss = jnp.sum(1.25 * mse) / jnp.float32(Q)

    occ = occ_ref[...]
    tot = jnp.sum(occ, axis=0)
    uniq = jnp.sum((tot > 0).astype(jnp.float32), axis=1)
    perp = jnp.sum(uniq) / jnp.float32(Q)
    union = jnp.sum(tot, axis=0)
    used = jnp.sum((union > 0).astype(jnp.int32))

    loss_ref[0, 0] = loss
    perp_ref[0, 0] = perp
    used_ref[0, 0] = used


def _big_cdist(y, a2, cb0, b20):
    a2r = a2.reshape(NB, 1, BN)
    b2r = b20.reshape(1, 1, K)
    codes, rowd2 = pl.pallas_call(
        _cdist_code_kernel,
        grid=(NB,),
        in_specs=[
            pl.BlockSpec((BN, D), lambda i: (i, 0)),
            pl.BlockSpec((1, 1, BN), lambda i: (i, 0, 0)),
            pl.BlockSpec((K, D), lambda i: (0, 0)),
            pl.BlockSpec((1, 1, K), lambda i: (0, 0, 0)),
        ],
        out_specs=[
            pl.BlockSpec((1, 1, BN), lambda i: (i, 0, 0)),
            pl.BlockSpec((1, 1, BN), lambda i: (i, 0, 0)),
        ],
        out_shape=[
            jax.ShapeDtypeStruct((NB, 1, BN), jnp.int32),
            jax.ShapeDtypeStruct((NB, 1, BN), jnp.float32),
        ],
    )(y, a2r, cb0, b2r)
    return codes.reshape(N), rowd2.reshape(N)


def _mappings(codebooks, cbn):
    cbn3 = cbn.reshape(Q, 1, K)
    maps, md2 = pl.pallas_call(
        _mapping_kernel,
        grid=(Q - 1,),
        in_specs=[
            pl.BlockSpec((1, K, D), lambda q: (q, 0, 0)),
            pl.BlockSpec((1, K, D), lambda q: (q + 1, 0, 0)),
            pl.BlockSpec((1, 1, K), lambda q: (q, 0, 0)),
            pl.BlockSpec((1, 1, K), lambda q: (q + 1, 0, 0)),
        ],
        out_specs=[
            pl.BlockSpec((1, 1, K), lambda q: (q, 0, 0)),
            pl.BlockSpec((1, 1, K), lambda q: (q, 0, 0)),
        ],
        out_shape=[
            jax.ShapeDtypeStruct((Q - 1, 1, K), jnp.int32),
            jax.ShapeDtypeStruct((Q - 1, 1, K), jnp.float32),
        ],
    )(codebooks, codebooks, cbn3, cbn3)
    return maps.reshape(Q - 1, K), md2.reshape(Q - 1, K)


def _sc_chain(codes0, rowd2, maps, md2):
    mesh = plsc.VectorSubcoreMesh(core_axis_name="c", subcore_axis_name="s",
                                  num_cores=NC, num_subcores=NS)
    f = pl.kernel(
        _sc_chain_kernel,
        out_type=[
            jax.ShapeDtypeStruct(((Q - 1) * N,), jnp.int32),
            jax.ShapeDtypeStruct((NW * Q * K,), jnp.int32),
            jax.ShapeDtypeStruct((NW * Q * LANES,), jnp.float32),
        ],
        mesh=mesh,
        compiler_params=pltpu.CompilerParams(needs_layout_passes=False),
        scratch_types=[
            pltpu.VMEM((CHUNK,), jnp.int32),
            pltpu.VMEM((CHUNK,), jnp.float32),
            pltpu.VMEM((K,), jnp.int32),
            pltpu.VMEM((K,), jnp.int32),
            pltpu.VMEM((K,), jnp.int32),
            pltpu.VMEM((K,), jnp.float32),
            pltpu.VMEM((K,), jnp.float32),
            pltpu.VMEM((K,), jnp.float32),
            pltpu.VMEM((CHUNK,), jnp.int32),
            pltpu.VMEM((CHUNK,), jnp.int32),
            pltpu.VMEM((CHUNK,), jnp.int32),
            pltpu.VMEM((K,), jnp.int32),
            pltpu.VMEM((K,), jnp.int32),
            pltpu.VMEM((K,), jnp.int32),
            pltpu.VMEM((K,), jnp.int32),
            pltpu.VMEM((Q * LANES,), jnp.float32),
        ],
    )
    codesq, occ, accs = f(codes0, rowd2, maps.reshape(-1), md2.reshape(-1))
    return (codesq.reshape(Q - 1, N), occ.reshape(NW, Q, K),
            accs.reshape(NW, Q, LANES))


def _scalars(accs, occ):
    loss, perp, used = pl.pallas_call(
        _scalars_kernel,
        out_specs=[
            pl.BlockSpec(memory_space=pltpu.SMEM),
            pl.BlockSpec(memory_space=pltpu.SMEM),
            pl.BlockSpec(memory_space=pltpu.SMEM),
        ],
        out_shape=[
            jax.ShapeDtypeStruct((1, 1), jnp.float32),
            jax.ShapeDtypeStruct((1, 1), jnp.float32),
            jax.ShapeDtypeStruct((1, 1), jnp.int32),
        ],
    )(accs, occ)
    return loss, perp, used


def kernel(x, W, b, codebooks):
    B, C, H, Wd = x.shape
    x_flat = jnp.transpose(x, (0, 2, 3, 1)).reshape(-1, C)
    y = x_flat @ W + b
    a2 = jnp.sum(y * y, axis=1)
    y, a2 = lax.optimization_barrier((y, a2))
    cbn = jnp.sum(codebooks * codebooks, axis=2)

    codes0, rowd2 = _big_cdist(y, a2, codebooks[0], cbn[0])
    maps, md2 = _mappings(codebooks, cbn)
    codesq, occ, accs = _sc_chain(codes0, rowd2, maps, md2)
    loss, perp, used = _scalars(accs, occ)

    codes = jnp.concatenate([codes0[None, :], codesq], axis=0)
    codes = codes.reshape(Q, B, H, Wd)
    return (codes, loss.reshape(()), perp.reshape(()),
            used.reshape(()).astype(jnp.int32))
